# baseline (speedup 1.0000x reference)
import functools

import jax
import jax.numpy as jnp
import ml_dtypes
import numpy as np
from jax import lax
from jax.experimental import pallas as pl
from jax.experimental.pallas import tpu as pltpu
from jax.experimental.pallas import tpu_sc as plsc

IMG_SIZE = 800
RADIUS = 0.003
ZNEAR = 0.01

B = 4
N = 100000
NPAD = 106496
ROWS = NPAD // 128
RT = B * ROWS
RB = 32
BLOCKS_PER_BATCH = ROWS // RB
NK = 9

PIX = IMG_SIZE * IMG_SIZE
IMG_LOCAL = 2 * PIX
NC, NS = 2, 16

ROWS_PER_TILE = RT // NC // NS
TILE_WORDS = PIX * 2 // NS
ZCH = 4000


def _camera():
    dist = 20.0
    elev = np.deg2rad(10.0)
    azim = 0.0
    cam = np.array([dist * np.cos(elev) * np.sin(azim),
                    dist * np.sin(elev),
                    dist * np.cos(elev) * np.cos(azim)], dtype=np.float64)
    at = np.zeros(3)
    up = np.array([0.0, 1.0, 0.0])
    z = at - cam
    z = z / np.linalg.norm(z)
    x = np.cross(up, z)
    x = x / np.linalg.norm(x)
    y = np.cross(z, x)
    R = np.stack([x, y, z], axis=1)
    T = -(R.T @ cam)
    return np.asarray(R, np.float32), np.asarray(T, np.float32)


_R, _T = _camera()
_RBF = np.asarray(_R, ml_dtypes.bfloat16).astype(np.float32)
_HALF = (IMG_SIZE - 1) / 2.0
_NDC_PER_PIX = 1.0 / _HALF
_R2 = RADIUS * RADIUS


def _project_body(xs_ref, ys_ref, zs_ref, idx_ref, val_ref):
    i = pl.program_id(0)
    x = xs_ref[...].astype(jnp.bfloat16).astype(jnp.float32)
    y = ys_ref[...].astype(jnp.bfloat16).astype(jnp.float32)
    z = zs_ref[...].astype(jnp.bfloat16).astype(jnp.float32)
    xv0 = x * _RBF[0, 0] + y * _RBF[1, 0] + z * _RBF[2, 0] + _T[0]
    xv1 = x * _RBF[0, 1] + y * _RBF[1, 1] + z * _RBF[2, 1] + _T[1]
    xv2 = x * _RBF[0, 2] + y * _RBF[1, 2] + z * _RBF[2, 2] + _T[2]
    px = _HALF * (1.0 - xv0)
    py = _HALF * (1.0 - xv1)
    cx = jnp.round(px).astype(jnp.int32)
    cy = jnp.round(py).astype(jnp.int32)

    b = i // BLOCKS_PER_BATCH
    img_off = (b % 2) * PIX
    row_in_batch = (i % BLOCKS_PER_BATCH) * RB + lax.broadcasted_iota(
        jnp.int32, (RB, 128), 0)
    e = row_in_batch * 128 + lax.broadcasted_iota(jnp.int32, (RB, 128), 1)
    valid = (xv2 > ZNEAR) & (e < N)

    k = 0
    for dy in (-1, 0, 1):
        iy = cy + dy
        pcy = 1.0 - iy.astype(jnp.float32) * _NDC_PER_PIX
        dy2 = (pcy - xv1) ** 2
        oky = (iy >= 0) & (iy < IMG_SIZE)
        row_base = jnp.clip(iy, 0, IMG_SIZE - 1) * IMG_SIZE + img_off
        for dx in (-1, 0, 1):
            ix = cx + dx
            pcx = 1.0 - ix.astype(jnp.float32) * _NDC_PER_PIX
            d2 = (pcx - xv0) ** 2 + dy2
            alpha = jnp.clip(1.0 - d2 / _R2, 0.0, 0.999)
            inb = oky & (ix >= 0) & (ix < IMG_SIZE) & valid & (d2 < _R2)
            contrib = jnp.where(inb, jnp.log1p(-alpha), 0.0)
            pos = row_base + jnp.clip(ix, 0, IMG_SIZE - 1)
            idx_ref[k] = pos
            val_ref[k] = contrib
            k += 1


def _project(xs, ys, zs):
    return pl.pallas_call(
        _project_body,
        grid=(RT // RB,),
        in_specs=[pl.BlockSpec((RB, 128), lambda i: (i, 0))] * 3,
        out_specs=[pl.BlockSpec((NK, RB, 128), lambda i: (0, i, 0))] * 2,
        out_shape=[
            jax.ShapeDtypeStruct((NK, RT, 128), jnp.int32),
            jax.ShapeDtypeStruct((NK, RT, 128), jnp.float32),
        ],
    )(xs, ys, zs)


def _splat_body(idx_hbm, val_hbm, out_hbm, image, idx_in, val_in, zbuf, obuf):
    c = lax.axis_index("c")
    s = lax.axis_index("s")

    def _zero16(i, carry):
        zbuf[pl.ds(i * 16, 16)] = jnp.zeros((16,), jnp.float32)
        return carry

    lax.fori_loop(0, ZCH // 16, _zero16, 0)
    for zc in range(TILE_WORDS // ZCH):
        pltpu.sync_copy(zbuf, image.at[pl.ds(s * TILE_WORDS + zc * ZCH, ZCH)])
    plsc.subcore_barrier()

    row0 = c * (RT // NC) + s * ROWS_PER_TILE
    for k in range(NK):
        pltpu.sync_copy(idx_hbm.at[k].at[pl.ds(row0, ROWS_PER_TILE)], idx_in)
        pltpu.sync_copy(val_hbm.at[k].at[pl.ds(row0, ROWS_PER_TILE)], val_in)

        def _scat(r, carry):
            pltpu.sync_copy(val_in.at[r],
                            image.at[idx_in.at[r]], add=True)
            return carry

        lax.fori_loop(0, ROWS_PER_TILE, _scat, 0)
    plsc.subcore_barrier()

    base = c * IMG_LOCAL + s * TILE_WORDS
    for zc in range(TILE_WORDS // ZCH):
        pltpu.sync_copy(image.at[pl.ds(s * TILE_WORDS + zc * ZCH, ZCH)], obuf)

        def _exp16(i, carry):
            v = obuf[pl.ds(i * 16, 16)]
            obuf[pl.ds(i * 16, 16)] = 1.0 - jnp.exp(v)
            return carry

        lax.fori_loop(0, ZCH // 16, _exp16, 0)
        pltpu.sync_copy(obuf, out_hbm.at[pl.ds(base + zc * ZCH, ZCH)])


@functools.cache
def _splat():
    return pl.kernel(
        _splat_body,
        out_type=jax.ShapeDtypeStruct((B * PIX,), jnp.float32),
        mesh=plsc.VectorSubcoreMesh(
            core_axis_name="c", subcore_axis_name="s", num_cores=NC,
            num_subcores=NS),
        scratch_types=[
            pltpu.VMEM_SHARED((IMG_LOCAL,), jnp.float32),
            pltpu.VMEM((ROWS_PER_TILE, 128), jnp.int32),
            pltpu.VMEM((ROWS_PER_TILE, 128), jnp.float32),
            pltpu.VMEM((ZCH,), jnp.float32),
            pltpu.VMEM((ZCH,), jnp.float32),
        ],
    )


def kernel(points):
    p = jnp.pad(points, ((0, 0), (0, NPAD - N), (0, 0)))
    xs = p[:, :, 0].reshape(RT, 128)
    ys = p[:, :, 1].reshape(RT, 128)
    zs = p[:, :, 2].reshape(RT, 128)
    idx, val = _project(xs, ys, zs)
    flat = _splat()(idx, val)
    a = flat.reshape(B, IMG_SIZE, IMG_SIZE)
    return jnp.broadcast_to(a[..., None], (B, IMG_SIZE, IMG_SIZE, 3))


# speedup vs baseline: 1.1283x; 1.1283x over previous
import functools

import jax
import jax.numpy as jnp
import ml_dtypes
import numpy as np
from jax import lax
from jax.experimental import pallas as pl
from jax.experimental.pallas import tpu as pltpu
from jax.experimental.pallas import tpu_sc as plsc

IMG_SIZE = 800
RADIUS = 0.003
ZNEAR = 0.01

B = 4
N = 100000
NPAD = 106496
ROWS = NPAD // 128
RT = B * ROWS
RB = 32
BLOCKS_PER_BATCH = ROWS // RB
NK = 9

PIX = IMG_SIZE * IMG_SIZE
IMG_LOCAL = 2 * PIX
NC, NS = 2, 16

ROWS_PER_TILE = RT // NC // NS
TILE_WORDS = PIX * 2 // NS
ZCH = 4000


def _camera():
    dist = 20.0
    elev = np.deg2rad(10.0)
    azim = 0.0
    cam = np.array([dist * np.cos(elev) * np.sin(azim),
                    dist * np.sin(elev),
                    dist * np.cos(elev) * np.cos(azim)], dtype=np.float64)
    at = np.zeros(3)
    up = np.array([0.0, 1.0, 0.0])
    z = at - cam
    z = z / np.linalg.norm(z)
    x = np.cross(up, z)
    x = x / np.linalg.norm(x)
    y = np.cross(z, x)
    R = np.stack([x, y, z], axis=1)
    T = -(R.T @ cam)
    return np.asarray(R, np.float32), np.asarray(T, np.float32)


_R, _T = _camera()
_RBF = np.asarray(_R, ml_dtypes.bfloat16).astype(np.float32)
_HALF = (IMG_SIZE - 1) / 2.0
_NDC_PER_PIX = 1.0 / _HALF
_R2 = RADIUS * RADIUS


def _project_body(xs_ref, ys_ref, zs_ref, idx_ref, val_ref):
    i = pl.program_id(0)
    x = xs_ref[...].astype(jnp.bfloat16).astype(jnp.float32)
    y = ys_ref[...].astype(jnp.bfloat16).astype(jnp.float32)
    z = zs_ref[...].astype(jnp.bfloat16).astype(jnp.float32)
    xv0 = x * _RBF[0, 0] + y * _RBF[1, 0] + z * _RBF[2, 0] + _T[0]
    xv1 = x * _RBF[0, 1] + y * _RBF[1, 1] + z * _RBF[2, 1] + _T[1]
    xv2 = x * _RBF[0, 2] + y * _RBF[1, 2] + z * _RBF[2, 2] + _T[2]
    px = _HALF * (1.0 - xv0)
    py = _HALF * (1.0 - xv1)
    cx = jnp.round(px).astype(jnp.int32)
    cy = jnp.round(py).astype(jnp.int32)

    b = i // BLOCKS_PER_BATCH
    img_off = (b % 2) * PIX
    row_in_batch = (i % BLOCKS_PER_BATCH) * RB + lax.broadcasted_iota(
        jnp.int32, (RB, 128), 0)
    e = row_in_batch * 128 + lax.broadcasted_iota(jnp.int32, (RB, 128), 1)
    valid = (xv2 > ZNEAR) & (e < N)

    k = 0
    for dy in (-1, 0, 1):
        iy = cy + dy
        pcy = 1.0 - iy.astype(jnp.float32) * _NDC_PER_PIX
        dy2 = (pcy - xv1) ** 2
        oky = (iy >= 0) & (iy < IMG_SIZE)
        row_base = jnp.clip(iy, 0, IMG_SIZE - 1) * IMG_SIZE + img_off
        for dx in (-1, 0, 1):
            ix = cx + dx
            pcx = 1.0 - ix.astype(jnp.float32) * _NDC_PER_PIX
            d2 = (pcx - xv0) ** 2 + dy2
            alpha = jnp.clip(1.0 - d2 / _R2, 0.0, 0.999)
            inb = oky & (ix >= 0) & (ix < IMG_SIZE) & valid & (d2 < _R2)
            contrib = jnp.where(inb, jnp.log1p(-alpha), 0.0)
            pos = row_base + jnp.clip(ix, 0, IMG_SIZE - 1)
            idx_ref[k] = pos
            val_ref[k] = contrib
            k += 1


def _project(xs, ys, zs):
    return pl.pallas_call(
        _project_body,
        grid=(RT // RB,),
        in_specs=[pl.BlockSpec((RB, 128), lambda i: (i, 0))] * 3,
        out_specs=[pl.BlockSpec((NK, RB, 128), lambda i: (0, i, 0))] * 2,
        out_shape=[
            jax.ShapeDtypeStruct((NK, RT, 128), jnp.int32),
            jax.ShapeDtypeStruct((NK, RT, 128), jnp.float32),
        ],
    )(xs, ys, zs)


def _splat_body(idx_hbm, val_hbm, out_hbm, image, idx_in, val_in, zbuf, obuf,
                in_sems, scat_sem):
    c = lax.axis_index("c")
    s = lax.axis_index("s")
    row0 = c * (RT // NC) + s * ROWS_PER_TILE
    nsub = 2 * NK
    HR = (56, 48)
    HOFF = (0, 56)

    def _load(j, p):
        h = j % 2
        r = row0 + HOFF[h]
        return (
            pltpu.async_copy(idx_hbm.at[j // 2].at[pl.ds(r, HR[h])],
                             idx_in.at[p].at[pl.ds(0, HR[h])], in_sems.at[p]),
            pltpu.async_copy(val_hbm.at[j // 2].at[pl.ds(r, HR[h])],
                             val_in.at[p].at[pl.ds(0, HR[h])], in_sems.at[p]),
        )

    loads = [_load(0, 0), _load(1, 1)]

    def _zero16(i, carry):
        zbuf[pl.ds(i * 16, 16)] = jnp.zeros((16,), jnp.float32)
        return carry

    lax.fori_loop(0, ZCH // 16, _zero16, 0)
    for zc in range(TILE_WORDS // ZCH):
        pltpu.sync_copy(zbuf, image.at[pl.ds(s * TILE_WORDS + zc * ZCH, ZCH)])
    plsc.subcore_barrier()

    for j in range(nsub):
        p = j % 2
        rows = HR[j % 2]
        for d in loads[j]:
            d.wait()
        ii, vi = idx_in.at[p], val_in.at[p]

        def _fire(r, carry):
            pltpu.async_copy(vi.at[r], image.at[ii.at[r]], scat_sem, add=True)
            return carry

        lax.fori_loop(0, rows, _fire, 0)
        pltpu.make_async_copy(
            val_hbm.at[j // 2].at[pl.ds(row0 + HOFF[j % 2], rows)],
            val_in.at[p].at[pl.ds(0, rows)], scat_sem).wait()
        if j + 2 < nsub:
            loads.append(_load(j + 2, p))
    plsc.subcore_barrier()

    base = c * IMG_LOCAL + s * TILE_WORDS
    for zc in range(TILE_WORDS // ZCH):
        pltpu.sync_copy(image.at[pl.ds(s * TILE_WORDS + zc * ZCH, ZCH)], obuf)

        def _exp16(i, carry):
            v = obuf[pl.ds(i * 16, 16)]
            obuf[pl.ds(i * 16, 16)] = 1.0 - jnp.exp(v)
            return carry

        lax.fori_loop(0, ZCH // 16, _exp16, 0)
        pltpu.sync_copy(obuf, out_hbm.at[pl.ds(base + zc * ZCH, ZCH)])


@functools.cache
def _splat():
    return pl.kernel(
        _splat_body,
        out_type=jax.ShapeDtypeStruct((B * PIX,), jnp.float32),
        mesh=plsc.VectorSubcoreMesh(
            core_axis_name="c", subcore_axis_name="s", num_cores=NC,
            num_subcores=NS),
        scratch_types=[
            pltpu.VMEM_SHARED((IMG_LOCAL,), jnp.float32),
            pltpu.VMEM((2, 56, 128), jnp.int32),
            pltpu.VMEM((2, 56, 128), jnp.float32),
            pltpu.VMEM((ZCH,), jnp.float32),
            pltpu.VMEM((ZCH,), jnp.float32),
            pltpu.SemaphoreType.DMA((2,)),
            pltpu.SemaphoreType.DMA,
        ],
    )


def kernel(points):
    p = jnp.pad(points, ((0, 0), (0, NPAD - N), (0, 0)))
    xs = p[:, :, 0].reshape(RT, 128)
    ys = p[:, :, 1].reshape(RT, 128)
    zs = p[:, :, 2].reshape(RT, 128)
    idx, val = _project(xs, ys, zs)
    flat = _splat()(idx, val)
    a = flat.reshape(B, IMG_SIZE, IMG_SIZE)
    return jnp.broadcast_to(a[..., None], (B, IMG_SIZE, IMG_SIZE, 3))


# speedup vs baseline: 1.1942x; 1.0584x over previous
import functools

import jax
import jax.numpy as jnp
import ml_dtypes
import numpy as np
from jax import lax
from jax.experimental import pallas as pl
from jax.experimental.pallas import tpu as pltpu
from jax.experimental.pallas import tpu_sc as plsc

IMG_SIZE = 800
RADIUS = 0.003
ZNEAR = 0.01

B = 4
N = 100000
NPAD = 106496
ROWS = NPAD // 128
RT = B * ROWS
RB = 32
BLOCKS_PER_BATCH = ROWS // RB
NK = 9

PIX = IMG_SIZE * IMG_SIZE
IMG_LOCAL = 2 * PIX
NC, NS = 2, 16

ROWS_PER_TILE = RT // NC // NS
TILE_WORDS = PIX * 2 // NS
ZCH = 3200


def _camera():
    dist = 20.0
    elev = np.deg2rad(10.0)
    azim = 0.0
    cam = np.array([dist * np.cos(elev) * np.sin(azim),
                    dist * np.sin(elev),
                    dist * np.cos(elev) * np.cos(azim)], dtype=np.float64)
    at = np.zeros(3)
    up = np.array([0.0, 1.0, 0.0])
    z = at - cam
    z = z / np.linalg.norm(z)
    x = np.cross(up, z)
    x = x / np.linalg.norm(x)
    y = np.cross(z, x)
    R = np.stack([x, y, z], axis=1)
    T = -(R.T @ cam)
    return np.asarray(R, np.float32), np.asarray(T, np.float32)


_R, _T = _camera()
_RBF = np.asarray(_R, ml_dtypes.bfloat16).astype(np.float32)
_HALF = (IMG_SIZE - 1) / 2.0
_NDC_PER_PIX = 1.0 / _HALF
_R2 = RADIUS * RADIUS


def _project_body(xs_ref, ys_ref, zs_ref, idx_ref, val_ref):
    i = pl.program_id(0)
    x = xs_ref[...].astype(jnp.bfloat16).astype(jnp.float32)
    y = ys_ref[...].astype(jnp.bfloat16).astype(jnp.float32)
    z = zs_ref[...].astype(jnp.bfloat16).astype(jnp.float32)
    xv0 = x * _RBF[0, 0] + y * _RBF[1, 0] + z * _RBF[2, 0] + _T[0]
    xv1 = x * _RBF[0, 1] + y * _RBF[1, 1] + z * _RBF[2, 1] + _T[1]
    xv2 = x * _RBF[0, 2] + y * _RBF[1, 2] + z * _RBF[2, 2] + _T[2]
    px = _HALF * (1.0 - xv0)
    py = _HALF * (1.0 - xv1)
    cx = jnp.round(px).astype(jnp.int32)
    cy = jnp.round(py).astype(jnp.int32)

    b = i // BLOCKS_PER_BATCH
    img_off = (b % 2) * PIX
    row_in_batch = (i % BLOCKS_PER_BATCH) * RB + lax.broadcasted_iota(
        jnp.int32, (RB, 128), 0)
    e = row_in_batch * 128 + lax.broadcasted_iota(jnp.int32, (RB, 128), 1)
    valid = (xv2 > ZNEAR) & (e < N)

    k = 0
    for dy in (-1, 0, 1):
        iy = cy + dy
        pcy = 1.0 - iy.astype(jnp.float32) * _NDC_PER_PIX
        dy2 = (pcy - xv1) ** 2
        oky = (iy >= 0) & (iy < IMG_SIZE)
        row_base = jnp.clip(iy, 0, IMG_SIZE - 1) * IMG_SIZE + img_off
        for dx in (-1, 0, 1):
            ix = cx + dx
            pcx = 1.0 - ix.astype(jnp.float32) * _NDC_PER_PIX
            d2 = (pcx - xv0) ** 2 + dy2
            alpha = jnp.clip(1.0 - d2 / _R2, 0.0, 0.999)
            inb = oky & (ix >= 0) & (ix < IMG_SIZE) & valid & (d2 < _R2)
            contrib = jnp.where(inb, jnp.log1p(-alpha), 0.0)
            pos = row_base + jnp.clip(ix, 0, IMG_SIZE - 1)
            idx_ref[k] = pos
            val_ref[k] = contrib
            k += 1


def _project(xs, ys, zs):
    return pl.pallas_call(
        _project_body,
        grid=(RT // RB,),
        in_specs=[pl.BlockSpec((RB, 128), lambda i: (i, 0))] * 3,
        out_specs=[pl.BlockSpec((NK, RB, 128), lambda i: (0, i, 0))] * 2,
        out_shape=[
            jax.ShapeDtypeStruct((NK, RT, 128), jnp.int32),
            jax.ShapeDtypeStruct((NK, RT, 128), jnp.float32),
        ],
    )(xs, ys, zs)


def _splat_body(idx_hbm, val_hbm, out_hbm, image, idx_in, val_in, zbuf, obuf,
                in_sems, scat_sems, io_sems, zsem):
    c = lax.axis_index("c")
    s = lax.axis_index("s")
    row0 = c * (RT // NC) + s * ROWS_PER_TILE
    nsub = 2 * NK
    HR = (56, 48)
    HOFF = (0, 56)

    def _load(j):
        p = j % 3
        h = j % 2
        r = row0 + HOFF[h]
        return (
            pltpu.async_copy(idx_hbm.at[j // 2].at[pl.ds(r, HR[h])],
                             idx_in.at[p].at[pl.ds(0, HR[h])], in_sems.at[p]),
            pltpu.async_copy(val_hbm.at[j // 2].at[pl.ds(r, HR[h])],
                             val_in.at[p].at[pl.ds(0, HR[h])], in_sems.at[p]),
        )

    loads = {0: _load(0), 1: _load(1)}

    def _zero16(i, carry):
        zbuf[pl.ds(i * 16, 16)] = jnp.zeros((16,), jnp.float32)
        return carry

    lax.fori_loop(0, ZCH // 16, _zero16, 0)
    zdescs = [
        pltpu.async_copy(zbuf, image.at[pl.ds(s * TILE_WORDS + zc * ZCH, ZCH)],
                         zsem)
        for zc in range(TILE_WORDS // ZCH)
    ]
    for d in zdescs:
        d.wait()
    plsc.subcore_barrier()

    def _drain_scat(j):
        pltpu.make_async_copy(
            val_hbm.at[j // 2].at[pl.ds(row0 + HOFF[j % 2], HR[j % 2])],
            val_in.at[j % 3].at[pl.ds(0, HR[j % 2])],
            scat_sems.at[j % 3]).wait()

    for j in range(nsub):
        p = j % 3
        for d in loads[j]:
            d.wait()
        ii, vi = idx_in.at[p], val_in.at[p]
        sem = scat_sems.at[p]

        def _fire(r, carry):
            pltpu.async_copy(vi.at[r], image.at[ii.at[r]], sem, add=True)
            return carry

        lax.fori_loop(0, HR[j % 2], _fire, 0)
        if j >= 1:
            _drain_scat(j - 1)
        if j + 2 < nsub:
            loads[j + 2] = _load(j + 2)
    _drain_scat(nsub - 1)
    plsc.subcore_barrier()

    base = c * IMG_LOCAL + s * TILE_WORDS
    nz = TILE_WORDS // ZCH
    wbuf = (zbuf, obuf)

    def _img_in(zc, b):
        return pltpu.async_copy(
            image.at[pl.ds(s * TILE_WORDS + zc * ZCH, ZCH)], wbuf[b],
            io_sems.at[b])

    indescs = {0: _img_in(0, 0)}
    outdescs = {}
    for zc in range(nz):
        b = zc % 2
        indescs[zc].wait()
        if zc + 1 < nz:
            if zc >= 1:
                outdescs[zc - 1].wait()
            indescs[zc + 1] = _img_in(zc + 1, 1 - b)
        w = wbuf[b]

        def _exp16(i, carry):
            v = w[pl.ds(i * 16, 16)]
            w[pl.ds(i * 16, 16)] = 1.0 - jnp.exp(v)
            return carry

        lax.fori_loop(0, ZCH // 16, _exp16, 0)
        outdescs[zc] = pltpu.async_copy(
            w, out_hbm.at[pl.ds(base + zc * ZCH, ZCH)], io_sems.at[b])
    outdescs[nz - 2].wait()
    outdescs[nz - 1].wait()


@functools.cache
def _splat():
    return pl.kernel(
        _splat_body,
        out_type=jax.ShapeDtypeStruct((B * PIX,), jnp.float32),
        mesh=plsc.VectorSubcoreMesh(
            core_axis_name="c", subcore_axis_name="s", num_cores=NC,
            num_subcores=NS),
        scratch_types=[
            pltpu.VMEM_SHARED((IMG_LOCAL,), jnp.float32),
            pltpu.VMEM((3, 56, 128), jnp.int32),
            pltpu.VMEM((3, 56, 128), jnp.float32),
            pltpu.VMEM((ZCH,), jnp.float32),
            pltpu.VMEM((ZCH,), jnp.float32),
            pltpu.SemaphoreType.DMA((3,)),
            pltpu.SemaphoreType.DMA((3,)),
            pltpu.SemaphoreType.DMA((2,)),
            pltpu.SemaphoreType.DMA,
        ],
    )


def kernel(points):
    p = jnp.pad(points, ((0, 0), (0, NPAD - N), (0, 0)))
    xs = p[:, :, 0].reshape(RT, 128)
    ys = p[:, :, 1].reshape(RT, 128)
    zs = p[:, :, 2].reshape(RT, 128)
    idx, val = _project(xs, ys, zs)
    flat = _splat()(idx, val)
    a = flat.reshape(B, IMG_SIZE, IMG_SIZE)
    return jnp.broadcast_to(a[..., None], (B, IMG_SIZE, IMG_SIZE, 3))
